# Initial kernel scaffold; baseline (speedup 1.0000x reference)
import jax
import jax.numpy as jnp
from jax.experimental import pallas as pl


def kernel(x, pos, edge_index, edge_attr, params):
    raise NotImplementedError("write your pallas kernel here")


# speedup vs baseline: 5.1484x; 5.1484x over previous
import functools

import jax
import jax.numpy as jnp
from jax import lax
from jax.experimental import pallas as pl
from jax.experimental.pallas import tpu as pltpu
from jax.experimental.pallas import tpu_sc as plsc

H = 128
N_NODES = 10000
NP = 10240
E = 320000
EP = 327680
EBLK = 1280
NBLK = 1024
NCORES = 2
NSUB = 16
CHUNK = 128
NCHUNKS = EP // CHUNK
CHUNKS_PER_TILE = NCHUNKS // (NCORES * NSUB)
ACC_ROWS_PER_TILE = NP // NSUB

_PREC = lax.Precision.DEFAULT


def _dot(a, b):
    return lax.dot_general(a, b, (((1,), (0,)), ((), ())),
                           precision=_PREC, preferred_element_type=jnp.float32)


def _ln(x, g, b):
    mu = jnp.mean(x, axis=-1, keepdims=True)
    xc = x - mu
    var = jnp.mean(xc * xc, axis=-1, keepdims=True)
    return xc / jnp.sqrt(var + 1e-5) * g + b


def _full(shape):
    return pl.BlockSpec(shape, lambda i: tuple(0 for _ in shape))


def _row_spec(blk):
    return pl.BlockSpec((blk, H), lambda i: (i, 0))


def _node_enc_body(x_ref, emb_ref, w1a_ref, w1b_ref, b1_ref, w2_ref, b2_ref,
                   w3_ref, b3_ref, g_ref, be_ref, o_ref):
    w1_top = _dot(emb_ref[...], w1a_ref[...])
    w1 = jnp.concatenate([w1_top, w1b_ref[...]], axis=0)
    h = jnp.maximum(_dot(x_ref[...], w1) + b1_ref[...], 0.0)
    h = jnp.maximum(_dot(h, w2_ref[...]) + b2_ref[...], 0.0)
    h = _dot(h, w3_ref[...]) + b3_ref[...]
    o_ref[...] = _ln(h, g_ref[...], be_ref[...])


def _mlp3_body(x_ref, w1_ref, b1_ref, w2_ref, b2_ref, w3_ref, b3_ref,
               g_ref, be_ref, o_ref):
    h = jnp.maximum(_dot(x_ref[...], w1_ref[...]) + b1_ref[...], 0.0)
    h = jnp.maximum(_dot(h, w2_ref[...]) + b2_ref[...], 0.0)
    h = _dot(h, w3_ref[...]) + b3_ref[...]
    o_ref[...] = _ln(h, g_ref[...], be_ref[...])


def _mlp3_noln_body(x_ref, w1_ref, b1_ref, w2_ref, b2_ref, w3_ref, b3_ref,
                    o_ref):
    h = jnp.maximum(_dot(x_ref[...], w1_ref[...]) + b1_ref[...], 0.0)
    h = jnp.maximum(_dot(h, w2_ref[...]) + b2_ref[...], 0.0)
    o_ref[...] = _dot(h, w3_ref[...]) + b3_ref[...]


def _ab_body(n_ref, wi_ref, wj_ref, a_ref, b_ref):
    a_ref[...] = _dot(n_ref[...], wi_ref[...])
    b_ref[...] = _dot(n_ref[...], wj_ref[...])


def _edge_body(gd_ref, gs_ref, e_ref, w1e_ref, b1_ref, w2_ref, b2_ref,
               w3_ref, b3_ref, g_ref, be_ref, eo_ref, mo_ref):
    x = e_ref[...]
    h = gd_ref[...] + gs_ref[...] + _dot(x, w1e_ref[...]) + b1_ref[...]
    h = jnp.maximum(h, 0.0)
    h = jnp.maximum(_dot(h, w2_ref[...]) + b2_ref[...], 0.0)
    m = _ln(_dot(h, w3_ref[...]) + b3_ref[...], g_ref[...], be_ref[...])
    mo_ref[...] = m
    eo_ref[...] = x + m


def _node_upd_body(n_ref, a0_ref, a1_ref, wn_ref, wa_ref, b1_ref, w2_ref,
                   b2_ref, w3_ref, b3_ref, g_ref, be_ref, no_ref):
    x = n_ref[...]
    acc = a0_ref[...] + a1_ref[...]
    h = jnp.maximum(_dot(x, wn_ref[...]) + _dot(acc, wa_ref[...]) + b1_ref[...], 0.0)
    h = jnp.maximum(_dot(h, w2_ref[...]) + b2_ref[...], 0.0)
    no_ref[...] = x + _ln(_dot(h, w3_ref[...]) + b3_ref[...], g_ref[...], be_ref[...])


def _wspecs(shapes):
    return [_full(s) for s in shapes]


def _node_encode(feat, emb_p, w1a, w1b_p, b1, w2, b2, w3, b3, g, be):
    return pl.pallas_call(
        _node_enc_body,
        grid=(NP // NBLK,),
        in_specs=[_row_spec(NBLK)] + _wspecs([
            (16, 16), (16, H), (H - 16, H), (1, H), (H, H), (1, H),
            (H, H), (1, H), (1, H), (1, H)]),
        out_specs=_row_spec(NBLK),
        out_shape=jax.ShapeDtypeStruct((NP, H), jnp.float32),
    )(feat, emb_p, w1a, w1b_p, b1, w2, b2, w3, b3, g, be)


def _edge_encode(feat, w1, b1, w2, b2, w3, b3, g, be):
    return pl.pallas_call(
        _mlp3_body,
        grid=(EP // EBLK,),
        in_specs=[_row_spec(EBLK)] + _wspecs([
            (H, H), (1, H), (H, H), (1, H), (H, H), (1, H), (1, H), (1, H)]),
        out_specs=_row_spec(EBLK),
        out_shape=jax.ShapeDtypeStruct((EP, H), jnp.float32),
    )(feat, w1, b1, w2, b2, w3, b3, g, be)


def _ab_project(node, wi, wj):
    return pl.pallas_call(
        _ab_body,
        grid=(NP // NBLK,),
        in_specs=[_row_spec(NBLK)] + _wspecs([(H, H), (H, H)]),
        out_specs=[_row_spec(NBLK), _row_spec(NBLK)],
        out_shape=[jax.ShapeDtypeStruct((NP, H), jnp.float32)] * 2,
    )(node, wi, wj)


def _edge_mlp(gd, gs, e, w1e, b1, w2, b2, w3, b3, g, be):
    return pl.pallas_call(
        _edge_body,
        grid=(EP // EBLK,),
        in_specs=[_row_spec(EBLK)] * 3 + _wspecs([
            (H, H), (1, H), (H, H), (1, H), (H, H), (1, H), (1, H), (1, H)]),
        out_specs=[_row_spec(EBLK), _row_spec(EBLK)],
        out_shape=[jax.ShapeDtypeStruct((EP, H), jnp.float32)] * 2,
    )(gd, gs, e, w1e, b1, w2, b2, w3, b3, g, be)


def _node_update(node, acc0, acc1, wn, wa, b1, w2, b2, w3, b3, g, be):
    return pl.pallas_call(
        _node_upd_body,
        grid=(NP // NBLK,),
        in_specs=[_row_spec(NBLK)] * 3 + _wspecs([
            (H, H), (H, H), (1, H), (H, H), (1, H), (H, H), (1, H),
            (1, H), (1, H)]),
        out_specs=_row_spec(NBLK),
        out_shape=jax.ShapeDtypeStruct((NP, H), jnp.float32),
    )(node, acc0, acc1, wn, wa, b1, w2, b2, w3, b3, g, be)


def _decode(node, w1, b1, w2, b2, w3, b3):
    return pl.pallas_call(
        _mlp3_noln_body,
        grid=(NP // NBLK,),
        in_specs=[_row_spec(NBLK)] + _wspecs([
            (H, H), (1, H), (H, H), (1, H), (H, H), (1, H)]),
        out_specs=_row_spec(NBLK),
        out_shape=jax.ShapeDtypeStruct((NP, H), jnp.float32),
    )(node, w1, b1, w2, b2, w3, b3)


def _sc_mesh():
    return plsc.VectorSubcoreMesh(core_axis_name="core",
                                  subcore_axis_name="subcore")


def _sc_gather(a, b, dst2d, src2d):
    out_t = (jax.ShapeDtypeStruct((EP, H), jnp.float32),
             jax.ShapeDtypeStruct((EP, H), jnp.float32))

    @functools.partial(pl.kernel, out_type=out_t, mesh=_sc_mesh())
    def k(a_hbm, b_hbm, d_hbm, s_hbm, gd_hbm, gs_hbm):
        def body(d_v, s_v, gd_v, gs_v):
            pltpu.sync_copy(a_hbm.at[d_v.at[0]], gd_v)
            pltpu.sync_copy(b_hbm.at[s_v.at[0]], gs_v)

        pltpu.emit_pipeline(
            body,
            grid=(NCHUNKS,),
            in_specs=[pl.BlockSpec((1, CHUNK), lambda i: (i, 0)),
                      pl.BlockSpec((1, CHUNK), lambda i: (i, 0))],
            out_specs=[pl.BlockSpec((CHUNK, H), lambda i: (i, 0)),
                       pl.BlockSpec((CHUNK, H), lambda i: (i, 0))],
            core_axis_name=("core", "subcore"),
            dimension_semantics=(pltpu.PARALLEL,),
        )(d_hbm, s_hbm, gd_hbm, gs_hbm)

    return k(a, b, dst2d, src2d)


def _sc_scatter(msg, dst2d, zeros):
    @functools.partial(
        pl.kernel,
        out_type=jax.ShapeDtypeStruct((NCORES, NP, H), jnp.float32),
        mesh=_sc_mesh(),
        scratch_types=[
            pltpu.VMEM((CHUNK, H), jnp.float32),
            pltpu.VMEM((1, CHUNK), jnp.int32),
            pltpu.VMEM_SHARED((NP, H), jnp.float32),
        ],
    )
    def k(m_hbm, d_hbm, z_hbm, o_hbm, m_v, i_v, acc):
        cid = lax.axis_index("core")
        sid = lax.axis_index("subcore")

        @pl.loop(0, ACC_ROWS_PER_TILE // CHUNK)
        def _zero(j):
            pltpu.sync_copy(
                z_hbm, acc.at[pl.ds(sid * ACC_ROWS_PER_TILE + j * CHUNK, CHUNK)])

        plsc.subcore_barrier()

        base = (cid * NSUB + sid) * CHUNKS_PER_TILE

        @pl.loop(0, CHUNKS_PER_TILE)
        def _scat(j):
            c = base + j
            pltpu.sync_copy(m_hbm.at[pl.ds(c * CHUNK, CHUNK)], m_v)
            pltpu.sync_copy(d_hbm.at[pl.ds(c, 1)], i_v)
            pltpu.sync_copy(m_v, acc.at[i_v.at[0]], add=True)

        plsc.subcore_barrier()

        @pl.loop(0, ACC_ROWS_PER_TILE // CHUNK)
        def _drain(j):
            r = sid * ACC_ROWS_PER_TILE + j * CHUNK
            pltpu.sync_copy(acc.at[pl.ds(r, CHUNK)],
                            o_hbm.at[cid, pl.ds(r, CHUNK)])

    return k(msg, dst2d, zeros)


def kernel(x, pos, edge_index, edge_attr, params):
    f32 = jnp.float32
    x = x.astype(jnp.int32)
    ei = edge_index.astype(jnp.int32)
    src, dst = ei[0], ei[1]
    padidx = jnp.full((EP - E,), N_NODES, jnp.int32)
    dst2d = jnp.concatenate([dst, padidx]).reshape(NCHUNKS, CHUNK)
    src2d = jnp.concatenate([src, padidx]).reshape(NCHUNKS, CHUNK)

    def b2d(b):
        return b.reshape(1, -1)

    def pad_lane(a, n):
        return jnp.pad(a, ((0, 0), (0, n - a.shape[1])))

    oh = (x[:, None] == jnp.arange(9, dtype=jnp.int32)[None, :]).astype(f32)
    feat = jnp.concatenate([oh, jnp.zeros((N_NODES, 7), f32), pos], axis=1)
    feat = jnp.pad(feat, ((0, NP - N_NODES), (0, H - feat.shape[1])))

    pni = params["node_in"]
    (w1, b1), (w2, b2), (w3, b3) = pni["lin"]
    g, be = pni["ln"]
    emb_p = jnp.pad(params["embed"], ((0, 7), (0, 0)))
    w1a = w1[:16]
    w1b_p = jnp.pad(w1[16:30], ((0, (H - 16) - 14), (0, 0)))
    node = _node_encode(feat, emb_p, w1a, w1b_p, b2d(b1), w2, b2d(b2),
                        w3, b2d(b3), b2d(g), b2d(be))

    pei = params["edge_in"]
    (w1, b1), (w2, b2), (w3, b3) = pei["lin"]
    g, be = pei["ln"]
    ef = jnp.pad(edge_attr, ((0, EP - E), (0, H - edge_attr.shape[1])))
    w1_p = jnp.pad(w1, ((0, H - w1.shape[0]), (0, 0)))
    edge = _edge_encode(ef, w1_p, b2d(b1), w2, b2d(b2), w3, b2d(b3),
                        b2d(g), b2d(be))

    zeros = jnp.zeros((CHUNK, H), f32)

    for layer in params["mp"]:
        (we1, eb1), (we2, eb2), (we3, eb3) = layer["lin_edge"]["lin"]
        eg, ebe = layer["lin_edge"]["ln"]
        w1i, w1j, w1e = we1[:H], we1[H:2 * H], we1[2 * H:]
        a, b = _ab_project(node, w1i, w1j)
        gd, gs = _sc_gather(a, b, dst2d, src2d)
        edge, msg = _edge_mlp(gd, gs, edge, w1e, b2d(eb1), we2, b2d(eb2),
                              we3, b2d(eb3), b2d(eg), b2d(ebe))
        accs = _sc_scatter(msg, dst2d, zeros)
        (wn1, nb1), (wn2, nb2), (wn3, nb3) = layer["lin_node"]["lin"]
        ng, nbe = layer["lin_node"]["ln"]
        node = _node_update(node, accs[0], accs[1], wn1[:H], wn1[H:],
                            b2d(nb1), wn2, b2d(nb2), wn3, b2d(nb3),
                            b2d(ng), b2d(nbe))

    pno = params["node_out"]
    (w1, b1), (w2, b2), (w3, b3) = pno["lin"]
    w3_p = pad_lane(w3, H)
    b3_p = jnp.pad(b3, (0, H - b3.shape[0]))
    out = _decode(node, w1, b2d(b1), w2, b2d(b2), w3_p, b2d(b3_p))
    return out[:N_NODES, :2]


# speedup vs baseline: 7.4405x; 1.4452x over previous
import functools

import jax
import jax.numpy as jnp
from jax import lax
from jax.experimental import pallas as pl
from jax.experimental.pallas import tpu as pltpu
from jax.experimental.pallas import tpu_sc as plsc

H = 128
N_NODES = 10000
NP = 10240
E = 320000
EP = 327680
EBLK = 1280
NBLK = 1024
NCORES = 2
NSUB = 16
CHUNK = 128
NCHUNKS = EP // CHUNK
CHUNKS_PER_TILE = NCHUNKS // (NCORES * NSUB)
ACC_ROWS_PER_TILE = NP // NSUB

_PREC = lax.Precision.DEFAULT


def _dot(a, b):
    return lax.dot_general(a, b, (((1,), (0,)), ((), ())),
                           precision=_PREC, preferred_element_type=jnp.float32)


def _ln(x, g, b):
    mu = jnp.mean(x, axis=-1, keepdims=True)
    xc = x - mu
    var = jnp.mean(xc * xc, axis=-1, keepdims=True)
    return xc / jnp.sqrt(var + 1e-5) * g + b


def _full(shape):
    return pl.BlockSpec(shape, lambda i: tuple(0 for _ in shape))


def _row_spec(blk):
    return pl.BlockSpec((blk, H), lambda i: (i, 0))


def _node_enc_body(x_ref, emb_ref, w1a_ref, w1b_ref, b1_ref, w2_ref, b2_ref,
                   w3_ref, b3_ref, g_ref, be_ref, o_ref, ob_ref):
    w1_top = _dot(emb_ref[...], w1a_ref[...])
    w1 = jnp.concatenate([w1_top, w1b_ref[...]], axis=0)
    h = jnp.maximum(_dot(x_ref[...], w1) + b1_ref[...], 0.0)
    h = jnp.maximum(_dot(h, w2_ref[...]) + b2_ref[...], 0.0)
    h = _dot(h, w3_ref[...]) + b3_ref[...]
    o = _ln(h, g_ref[...], be_ref[...])
    o_ref[...] = o
    ob_ref[...] = o.astype(jnp.bfloat16)


def _mlp3_body(x_ref, w1_ref, b1_ref, w2_ref, b2_ref, w3_ref, b3_ref,
               g_ref, be_ref, o_ref):
    h = jnp.maximum(_dot(x_ref[...], w1_ref[...]) + b1_ref[...], 0.0)
    h = jnp.maximum(_dot(h, w2_ref[...]) + b2_ref[...], 0.0)
    h = _dot(h, w3_ref[...]) + b3_ref[...]
    o_ref[...] = _ln(h, g_ref[...], be_ref[...])


def _mlp3_noln_body(x_ref, w1_ref, b1_ref, w2_ref, b2_ref, w3_ref, b3_ref,
                    o_ref):
    h = jnp.maximum(_dot(x_ref[...], w1_ref[...]) + b1_ref[...], 0.0)
    h = jnp.maximum(_dot(h, w2_ref[...]) + b2_ref[...], 0.0)
    o_ref[...] = _dot(h, w3_ref[...]) + b3_ref[...]


def _ab_body(n_ref, wi_ref, wj_ref, a_ref, b_ref):
    a_ref[...] = _dot(n_ref[...], wi_ref[...])
    b_ref[...] = _dot(n_ref[...], wj_ref[...])


def _edge_body(gd_ref, gs_ref, e_ref, w1e_ref, b1_ref, w2_ref, b2_ref,
               w3_ref, b3_ref, g_ref, be_ref, eo_ref, mo_ref):
    x = e_ref[...]
    h = gd_ref[...] + gs_ref[...] + _dot(x, w1e_ref[...]) + b1_ref[...]
    h = jnp.maximum(h, 0.0)
    h = jnp.maximum(_dot(h, w2_ref[...]) + b2_ref[...], 0.0)
    m = _ln(_dot(h, w3_ref[...]) + b3_ref[...], g_ref[...], be_ref[...])
    mo_ref[...] = m
    eo_ref[...] = x + m


def _node_upd_body(n_ref, a0_ref, a1_ref, wn_ref, wa_ref, b1_ref, w2_ref,
                   b2_ref, w3_ref, b3_ref, g_ref, be_ref, no_ref, nb_ref):
    x = n_ref[...]
    acc = a0_ref[...] + a1_ref[...]
    h = jnp.maximum(_dot(x, wn_ref[...]) + _dot(acc, wa_ref[...]) + b1_ref[...], 0.0)
    h = jnp.maximum(_dot(h, w2_ref[...]) + b2_ref[...], 0.0)
    o = x + _ln(_dot(h, w3_ref[...]) + b3_ref[...], g_ref[...], be_ref[...])
    no_ref[...] = o
    nb_ref[...] = o.astype(jnp.bfloat16)


def _wspecs(shapes):
    return [_full(s) for s in shapes]


def _node_encode(feat, emb_p, w1a, w1b_p, b1, w2, b2, w3, b3, g, be):
    return pl.pallas_call(
        _node_enc_body,
        grid=(NP // NBLK,),
        in_specs=[_row_spec(NBLK)] + _wspecs([
            (16, 16), (16, H), (H - 16, H), (1, H), (H, H), (1, H),
            (H, H), (1, H), (1, H), (1, H)]),
        out_specs=[_row_spec(NBLK), _row_spec(NBLK)],
        out_shape=[jax.ShapeDtypeStruct((NP, H), jnp.float32),
                   jax.ShapeDtypeStruct((NP, H), jnp.bfloat16)],
    )(feat, emb_p, w1a, w1b_p, b1, w2, b2, w3, b3, g, be)


def _edge_encode(feat, w1, b1, w2, b2, w3, b3, g, be):
    return pl.pallas_call(
        _mlp3_body,
        grid=(EP // EBLK,),
        in_specs=[_row_spec(EBLK)] + _wspecs([
            (H, H), (1, H), (H, H), (1, H), (H, H), (1, H), (1, H), (1, H)]),
        out_specs=_row_spec(EBLK),
        out_shape=jax.ShapeDtypeStruct((EP, H), jnp.float32),
    )(feat, w1, b1, w2, b2, w3, b3, g, be)


def _ab_project(node, wi, wj):
    return pl.pallas_call(
        _ab_body,
        grid=(NP // NBLK,),
        in_specs=[_row_spec(NBLK)] + _wspecs([(H, H), (H, H)]),
        out_specs=[_row_spec(NBLK), _row_spec(NBLK)],
        out_shape=[jax.ShapeDtypeStruct((NP, H), jnp.float32)] * 2,
    )(node, wi, wj)


def _edge_mlp(gd, gs, e, w1e, b1, w2, b2, w3, b3, g, be):
    return pl.pallas_call(
        _edge_body,
        grid=(EP // EBLK,),
        in_specs=[_row_spec(EBLK)] * 3 + _wspecs([
            (H, H), (1, H), (H, H), (1, H), (H, H), (1, H), (1, H), (1, H)]),
        out_specs=[_row_spec(EBLK), _row_spec(EBLK)],
        out_shape=[jax.ShapeDtypeStruct((EP, H), jnp.float32)] * 2,
    )(gd, gs, e, w1e, b1, w2, b2, w3, b3, g, be)


def _node_update(node, acc0, acc1, wn, wa, b1, w2, b2, w3, b3, g, be):
    return pl.pallas_call(
        _node_upd_body,
        grid=(NP // NBLK,),
        in_specs=[_row_spec(NBLK)] * 3 + _wspecs([
            (H, H), (H, H), (1, H), (H, H), (1, H), (H, H), (1, H),
            (1, H), (1, H)]),
        out_specs=[_row_spec(NBLK), _row_spec(NBLK)],
        out_shape=[jax.ShapeDtypeStruct((NP, H), jnp.float32),
                   jax.ShapeDtypeStruct((NP, H), jnp.bfloat16)],
    )(node, acc0, acc1, wn, wa, b1, w2, b2, w3, b3, g, be)


def _decode(node, w1, b1, w2, b2, w3, b3):
    return pl.pallas_call(
        _mlp3_noln_body,
        grid=(NP // NBLK,),
        in_specs=[_row_spec(NBLK)] + _wspecs([
            (H, H), (1, H), (H, H), (1, H), (H, H), (1, H)]),
        out_specs=_row_spec(NBLK),
        out_shape=jax.ShapeDtypeStruct((NP, H), jnp.float32),
    )(node, w1, b1, w2, b2, w3, b3)


def _sc_mesh():
    return plsc.VectorSubcoreMesh(core_axis_name="core",
                                  subcore_axis_name="subcore")


def _sc_gather(a, b, dst2d, src2d):
    out_t = (jax.ShapeDtypeStruct((EP, H), jnp.float32),
             jax.ShapeDtypeStruct((EP, H), jnp.float32))

    @functools.partial(pl.kernel, out_type=out_t, mesh=_sc_mesh(),
                       scratch_types=[pltpu.SemaphoreType.DMA,
                                      pltpu.SemaphoreType.DMA])
    def k(a_hbm, b_hbm, d_hbm, s_hbm, gd_hbm, gs_hbm, sg0, sg1):
        def body(d_v, s_v, gd_v, gs_v):
            c0 = pltpu.async_copy(a_hbm.at[d_v.at[0]], gd_v, sg0)
            c1 = pltpu.async_copy(b_hbm.at[s_v.at[0]], gs_v, sg1)
            c0.wait()
            c1.wait()

        pltpu.emit_pipeline(
            body,
            grid=(NCHUNKS,),
            in_specs=[pl.BlockSpec((1, CHUNK), lambda i: (i, 0)),
                      pl.BlockSpec((1, CHUNK), lambda i: (i, 0))],
            out_specs=[pl.BlockSpec((CHUNK, H), lambda i: (i, 0)),
                       pl.BlockSpec((CHUNK, H), lambda i: (i, 0))],
            core_axis_name=("core", "subcore"),
            dimension_semantics=(pltpu.PARALLEL,),
        )(d_hbm, s_hbm, gd_hbm, gs_hbm)

    return k(a, b, dst2d, src2d)


def _sc_scatter(msg, dst2d, zeros):
    @functools.partial(
        pl.kernel,
        out_type=jax.ShapeDtypeStruct((NCORES, NP, H), jnp.float32),
        mesh=_sc_mesh(),
        scratch_types=[
            pltpu.VMEM((2, CHUNK, H), jnp.float32),
            pltpu.VMEM((2, CHUNK), jnp.int32),
            pltpu.VMEM_SHARED((NP, H), jnp.float32),
            pltpu.SemaphoreType.DMA,
            pltpu.SemaphoreType.DMA,
        ],
    )
    def k(m_hbm, d_hbm, z_hbm, o_hbm, m_v, i_v, acc, sem0, sem1):
        cid = lax.axis_index("core")
        sid = lax.axis_index("subcore")

        @pl.loop(0, ACC_ROWS_PER_TILE // CHUNK)
        def _zero(j):
            pltpu.sync_copy(
                z_hbm, acc.at[pl.ds(sid * ACC_ROWS_PER_TILE + j * CHUNK, CHUNK)])

        plsc.subcore_barrier()

        base = (cid * NSUB + sid) * CHUNKS_PER_TILE
        sems = (sem0, sem1)

        def load(c, slot, sem):
            pltpu.async_copy(m_hbm.at[pl.ds(c * CHUNK, CHUNK)],
                             m_v.at[slot], sem)
            pltpu.async_copy(d_hbm.at[pl.ds(c, 1)],
                             i_v.at[pl.ds(slot, 1)], sem)

        def drain_add(c, slot, sem):
            pltpu.make_async_copy(m_hbm.at[pl.ds(c * CHUNK, CHUNK)],
                                  m_v.at[slot], sem).wait()
            pltpu.make_async_copy(d_hbm.at[pl.ds(c, 1)],
                                  i_v.at[pl.ds(slot, 1)], sem).wait()
            pltpu.sync_copy(m_v.at[slot], acc.at[i_v.at[slot]], add=True)

        load(base, 0, sems[0])

        @pl.loop(0, CHUNKS_PER_TILE // 2)
        def _scat(t):
            c = base + 2 * t
            load(c + 1, 1, sems[1])
            drain_add(c, 0, sems[0])

            @pl.when(t < CHUNKS_PER_TILE // 2 - 1)
            def _():
                load(c + 2, 0, sems[0])

            drain_add(c + 1, 1, sems[1])

        plsc.subcore_barrier()

        @pl.loop(0, ACC_ROWS_PER_TILE // CHUNK)
        def _drain(j):
            r = sid * ACC_ROWS_PER_TILE + j * CHUNK
            pltpu.sync_copy(acc.at[pl.ds(r, CHUNK)],
                            o_hbm.at[cid, pl.ds(r, CHUNK)])

    return k(msg, dst2d, zeros)


def kernel(x, pos, edge_index, edge_attr, params):
    f32 = jnp.float32
    x = x.astype(jnp.int32)
    ei = edge_index.astype(jnp.int32)
    src, dst = ei[0], ei[1]
    padidx = jnp.full((EP - E,), N_NODES, jnp.int32)
    dst2d = jnp.concatenate([dst, padidx]).reshape(NCHUNKS, CHUNK)
    src2d = jnp.concatenate([src, padidx]).reshape(NCHUNKS, CHUNK)

    def b2d(b):
        return b.reshape(1, -1)

    def pad_lane(a, n):
        return jnp.pad(a, ((0, 0), (0, n - a.shape[1])))

    oh = (x[:, None] == jnp.arange(9, dtype=jnp.int32)[None, :]).astype(f32)
    feat = jnp.concatenate([oh, jnp.zeros((N_NODES, 7), f32), pos], axis=1)
    feat = jnp.pad(feat, ((0, NP - N_NODES), (0, H - feat.shape[1])))

    pni = params["node_in"]
    (w1, b1), (w2, b2), (w3, b3) = pni["lin"]
    g, be = pni["ln"]
    emb_p = jnp.pad(params["embed"], ((0, 7), (0, 0)))
    w1a = w1[:16]
    w1b_p = jnp.pad(w1[16:30], ((0, (H - 16) - 14), (0, 0)))
    node, node_bf = _node_encode(feat, emb_p, w1a, w1b_p, b2d(b1), w2, b2d(b2),
                                 w3, b2d(b3), b2d(g), b2d(be))

    pei = params["edge_in"]
    (w1, b1), (w2, b2), (w3, b3) = pei["lin"]
    g, be = pei["ln"]
    ef = jnp.pad(edge_attr, ((0, EP - E), (0, H - edge_attr.shape[1])))
    w1_p = jnp.pad(w1, ((0, H - w1.shape[0]), (0, 0)))
    edge = _edge_encode(ef, w1_p, b2d(b1), w2, b2d(b2), w3, b2d(b3),
                        b2d(g), b2d(be))

    zeros = jnp.zeros((CHUNK, H), f32)

    for layer in params["mp"]:
        (we1, eb1), (we2, eb2), (we3, eb3) = layer["lin_edge"]["lin"]
        eg, ebe = layer["lin_edge"]["ln"]
        w1i, w1j, w1e = we1[:H], we1[H:2 * H], we1[2 * H:]
        a, b = _ab_project(node, w1i, w1j)
        gd, gs = _sc_gather(a, b, dst2d, src2d)
        edge, msg = _edge_mlp(gd, gs, edge, w1e, b2d(eb1),
                              we2, b2d(eb2), we3, b2d(eb3), b2d(eg), b2d(ebe))
        accs = _sc_scatter(msg, dst2d, zeros)
        (wn1, nb1), (wn2, nb2), (wn3, nb3) = layer["lin_node"]["lin"]
        ng, nbe = layer["lin_node"]["ln"]
        node, node_bf = _node_update(node, accs[0], accs[1], wn1[:H], wn1[H:],
                                     b2d(nb1), wn2, b2d(nb2), wn3, b2d(nb3),
                                     b2d(ng), b2d(nbe))

    pno = params["node_out"]
    (w1, b1), (w2, b2), (w3, b3) = pno["lin"]
    w3_p = pad_lane(w3, H)
    b3_p = jnp.pad(b3, (0, H - b3.shape[0]))
    out = _decode(node, w1, b2d(b1), w2, b2d(b2), w3_p, b2d(b3_p))
    return out[:N_NODES, :2]
